# Initial kernel scaffold; baseline (speedup 1.0000x reference)
import jax
import jax.numpy as jnp
from jax.experimental import pallas as pl


def kernel(rois, mrcnn_class, mrcnn_bbox, image_meta):
    raise NotImplementedError("write your pallas kernel here")


# speedup vs baseline: 17.8819x; 17.8819x over previous
import jax
import jax.numpy as jnp
import numpy as np
from jax.experimental import pallas as pl

_MIN_CONF = 0.7
_NMS_THR = 0.3
_MAX_INST = 100
_ACC_CAP = 160
_NEG = float("-inf")


def _prep_kernel(probs_ref, deltas_ref, rois_ref, win_ref,
                 y1_ref, x1_ref, y2_ref, x2_ref, msc_ref, cid_ref):
    probs = probs_ref[0]
    tn, num_c = probs.shape
    maxv = jnp.max(probs, axis=1, keepdims=True)
    ci = jax.lax.broadcasted_iota(jnp.int32, (tn, num_c), 1)
    cid = jnp.min(jnp.where(probs == maxv, ci, num_c), axis=1,
                  keepdims=True)

    deltas = deltas_ref[0]
    col = jax.lax.broadcasted_iota(jnp.int32, (tn, 4 * num_c), 1)
    csel = (col >> 2) == cid
    j_of = col & 3
    d = [jnp.sum(jnp.where(csel & (j_of == j), deltas, 0.0), axis=1,
                 keepdims=True)
         for j in range(4)]
    dy, dx, dh, dw = d[0] * 0.1, d[1] * 0.1, d[2] * 0.2, d[3] * 0.2

    r = rois_ref[0]
    ry1, rx1, ry2, rx2 = r[:, 0:1], r[:, 1:2], r[:, 2:3], r[:, 3:4]
    height = ry2 - ry1
    width = rx2 - rx1
    cy = ry1 + 0.5 * height + dy * height
    cx = rx1 + 0.5 * width + dx * width
    h = height * jnp.exp(dh)
    w = width * jnp.exp(dw)
    y1 = cy - 0.5 * h
    x1 = cx - 0.5 * w
    y2 = y1 + h
    x2 = x1 + w
    wy1 = win_ref[0, 0, 0]
    wx1 = win_ref[0, 0, 1]
    wy2 = win_ref[0, 0, 2]
    wx2 = win_ref[0, 0, 3]
    y1_ref[0] = jnp.clip(y1, wy1, wy2)
    x1_ref[0] = jnp.clip(x1, wx1, wx2)
    y2_ref[0] = jnp.clip(y2, wy1, wy2)
    x2_ref[0] = jnp.clip(x2, wx1, wx2)
    elig = (cid > 0) & (maxv >= _MIN_CONF)
    msc_ref[0] = jnp.where(elig, maxv, _NEG)
    cid_ref[0] = cid


def _select_kernel(y1_ref, x1_ref, y2_ref, x2_ref, msc_ref, cid_ref, out_ref):
    rows, lanes = y1_ref.shape[1], y1_ref.shape[2]
    n = rows * lanes
    y1 = y1_ref[0]
    x1 = x1_ref[0]
    y2 = y2_ref[0]
    x2 = x2_ref[0]
    msc0 = msc_ref[0]
    cid = cid_ref[0]
    area = (y2 - y1) * (x2 - x1)

    iota_n = (jax.lax.broadcasted_iota(jnp.int32, (rows, lanes), 0) * lanes
              + jax.lax.broadcasted_iota(jnp.int32, (rows, lanes), 1))
    iota_cap = jax.lax.broadcasted_iota(jnp.int32, (1, _ACC_CAP), 1)
    iota_cls = jax.lax.broadcasted_iota(jnp.int32, (1, 128), 1)
    neg = jnp.float32(_NEG)

    def cond(st):
        msc, counts, aS, aY1, aX1, aY2, aX2, aA, aC, aR, nacc = st
        m = jnp.max(msc)
        kth = jnp.sum(jnp.where(iota_cap == _MAX_INST - 1, aS, 0.0))
        return (m > neg) & ((nacc < _MAX_INST)
                            | ((m >= kth) & (nacc < _ACC_CAP)))

    def body(st):
        msc, counts, aS, aY1, aX1, aY2, aX2, aA, aC, aR, nacc = st
        m = jnp.max(msc)
        i = jnp.min(jnp.where(msc == m, iota_n, n))
        pick = iota_n == i
        ci_ = jnp.sum(jnp.where(pick, cid, 0))
        by1 = jnp.sum(jnp.where(pick, y1, 0.0))
        bx1 = jnp.sum(jnp.where(pick, x1, 0.0))
        by2 = jnp.sum(jnp.where(pick, y2, 0.0))
        bx2 = jnp.sum(jnp.where(pick, x2, 0.0))
        ba = jnp.sum(jnp.where(pick, area, 0.0))

        yy1 = jnp.maximum(by1, aY1)
        xx1 = jnp.maximum(bx1, aX1)
        yy2 = jnp.minimum(by2, aY2)
        xx2 = jnp.minimum(bx2, aX2)
        inter = jnp.maximum(0.0, yy2 - yy1) * jnp.maximum(0.0, xx2 - xx1)
        union = ba + aA - inter
        iou = jnp.where(union > 0, inter / jnp.maximum(union, 1e-12), 0.0)
        samecls = (aC == ci_) & (iota_cap < nacc)
        suppressed = jnp.any(samecls & (iou > _NMS_THR))
        cnt_c = jnp.sum(jnp.where(iota_cls == ci_, counts, 0))
        accept = jnp.logical_and(jnp.logical_not(suppressed),
                                 cnt_c < _MAX_INST)

        slotmask = (iota_cap == nacc) & accept
        aS = jnp.where(slotmask, m, aS)
        aY1 = jnp.where(slotmask, by1, aY1)
        aX1 = jnp.where(slotmask, bx1, aX1)
        aY2 = jnp.where(slotmask, by2, aY2)
        aX2 = jnp.where(slotmask, bx2, aX2)
        aA = jnp.where(slotmask, ba, aA)
        aC = jnp.where(slotmask, ci_, aC)
        aR = jnp.where(slotmask, cnt_c, aR)
        counts = jnp.where((iota_cls == ci_) & accept, counts + 1, counts)
        nacc = nacc + accept.astype(jnp.int32)
        msc = jnp.where(pick, neg, msc)
        return (msc, counts, aS, aY1, aX1, aY2, aX2, aA, aC, aR, nacc)

    st0 = (
        msc0,
        jnp.zeros((1, 128), jnp.int32),
        jnp.full((1, _ACC_CAP), _NEG, jnp.float32),
        jnp.zeros((1, _ACC_CAP), jnp.float32),
        jnp.zeros((1, _ACC_CAP), jnp.float32),
        jnp.zeros((1, _ACC_CAP), jnp.float32),
        jnp.zeros((1, _ACC_CAP), jnp.float32),
        jnp.zeros((1, _ACC_CAP), jnp.float32),
        jnp.full((1, _ACC_CAP), -1, jnp.int32),
        jnp.zeros((1, _ACC_CAP), jnp.int32),
        jnp.int32(0),
    )
    st = jax.lax.while_loop(cond, body, st0)
    _, _, aS, aY1, aX1, aY2, aX2, aA, aC, aR, nacc = st

    gr = aC * _MAX_INST + aR
    rowi = jax.lax.broadcasted_iota(jnp.int32, (_MAX_INST, 6), 0)
    colj = jax.lax.broadcasted_iota(jnp.int32, (_MAX_INST, 6), 1)
    bigi = jnp.int32(1 << 30)

    def fbody(t, st2):
        det, avail_i = st2
        avail = avail_i != 0
        mm = jnp.max(jnp.where(avail, aS, neg))
        cand = avail & (aS == mm)
        g = jnp.min(jnp.where(cand, gr, bigi))
        slot = jnp.min(jnp.where(cand & (gr == g), iota_cap, _ACC_CAP))
        pickc = iota_cap == slot
        vy1 = jnp.sum(jnp.where(pickc, aY1, 0.0))
        vx1 = jnp.sum(jnp.where(pickc, aX1, 0.0))
        vy2 = jnp.sum(jnp.where(pickc, aY2, 0.0))
        vx2 = jnp.sum(jnp.where(pickc, aX2, 0.0))
        vc = jnp.sum(jnp.where(pickc, aC, 0)).astype(jnp.float32)
        vs = jnp.sum(jnp.where(pickc, aS, 0.0))
        ok = mm > neg
        rowm = (rowi == t) & ok
        rowvals = jnp.where(colj == 0, vy1,
                   jnp.where(colj == 1, vx1,
                    jnp.where(colj == 2, vy2,
                     jnp.where(colj == 3, vx2,
                      jnp.where(colj == 4, vc, vs)))))
        det = jnp.where(rowm, rowvals, det)
        avail_i = jnp.where(iota_cap != slot, avail_i, 0)
        return det, avail_i

    det0 = jnp.zeros((_MAX_INST, 6), jnp.float32)
    avail0 = (iota_cap < nacc).astype(jnp.int32)
    det, _ = jax.lax.fori_loop(0, _MAX_INST, fbody, (det0, avail0))
    out_ref[0] = det


def kernel(rois, mrcnn_class, mrcnn_bbox, image_meta):
    B, N, C = mrcnn_class.shape
    shift = jnp.asarray(np.array([0.0, 0.0, 1.0, 1.0], dtype=np.float32))
    image_shape = image_meta[0, 4:7]
    scale = jnp.concatenate([image_shape[:2], image_shape[:2]]) - 1.0
    windows = ((image_meta[:, 7:11] - shift) / scale).reshape(B, 1, 4)

    deltas2d = mrcnn_bbox.reshape(B, N, C * 4)

    T = 10
    assert N % T == 0 and N % 8 == 0
    TN = N // T
    N8 = N // 8

    outs = pl.pallas_call(
        _prep_kernel,
        grid=(B, T),
        in_specs=[
            pl.BlockSpec((1, TN, C), lambda b, t: (b, t, 0)),
            pl.BlockSpec((1, TN, C * 4), lambda b, t: (b, t, 0)),
            pl.BlockSpec((1, TN, 4), lambda b, t: (b, t, 0)),
            pl.BlockSpec((1, 1, 4), lambda b, t: (b, 0, 0)),
        ],
        out_specs=[
            pl.BlockSpec((1, TN, 1), lambda b, t: (b, t, 0)),
            pl.BlockSpec((1, TN, 1), lambda b, t: (b, t, 0)),
            pl.BlockSpec((1, TN, 1), lambda b, t: (b, t, 0)),
            pl.BlockSpec((1, TN, 1), lambda b, t: (b, t, 0)),
            pl.BlockSpec((1, TN, 1), lambda b, t: (b, t, 0)),
            pl.BlockSpec((1, TN, 1), lambda b, t: (b, t, 0)),
        ],
        out_shape=[
            jax.ShapeDtypeStruct((B, N, 1), jnp.float32),
            jax.ShapeDtypeStruct((B, N, 1), jnp.float32),
            jax.ShapeDtypeStruct((B, N, 1), jnp.float32),
            jax.ShapeDtypeStruct((B, N, 1), jnp.float32),
            jax.ShapeDtypeStruct((B, N, 1), jnp.float32),
            jax.ShapeDtypeStruct((B, N, 1), jnp.int32),
        ],
    )(mrcnn_class, deltas2d, rois, windows)
    y1, x1, y2, x2, msc, cid = [o.reshape(B, 8, N8) for o in outs]

    det = pl.pallas_call(
        _select_kernel,
        grid=(B,),
        in_specs=[
            pl.BlockSpec((1, 8, N8), lambda b: (b, 0, 0)),
            pl.BlockSpec((1, 8, N8), lambda b: (b, 0, 0)),
            pl.BlockSpec((1, 8, N8), lambda b: (b, 0, 0)),
            pl.BlockSpec((1, 8, N8), lambda b: (b, 0, 0)),
            pl.BlockSpec((1, 8, N8), lambda b: (b, 0, 0)),
            pl.BlockSpec((1, 8, N8), lambda b: (b, 0, 0)),
        ],
        out_specs=pl.BlockSpec((1, _MAX_INST, 6), lambda b: (b, 0, 0)),
        out_shape=jax.ShapeDtypeStruct((B, _MAX_INST, 6), jnp.float32),
    )(y1, x1, y2, x2, msc, cid)
    return det
